# baseline (speedup 1.0000x reference)
import functools

import jax
import jax.numpy as jnp
from jax import lax
from jax.experimental import pallas as pl
from jax.experimental.pallas import tpu as pltpu
from jax.experimental.pallas import tpu_sc as plsc

_D_MODEL = 128
_EPS = 1e-5


_NC, _NS = 2, 16
_NW = _NC * _NS
_CHUNK = 1280


def _sc_gather_body(n_per_w, table_hbm, idx_hbm, out_hbm, idx_v, rows_v, sem):
    wid = lax.axis_index("s") * _NC + lax.axis_index("c")
    base = wid * n_per_w
    for j in range(n_per_w // _CHUNK):
        off = base + j * _CHUNK
        pltpu.sync_copy(idx_hbm.at[pl.ds(off, _CHUNK)], idx_v)
        pltpu.async_copy(table_hbm.at[idx_v], rows_v, sem).wait()
        pltpu.sync_copy(rows_v, out_hbm.at[pl.ds(off, _CHUNK)])


def _sc_gather(table, idx):
    n = idx.shape[0]
    d = table.shape[1]
    n_per_w = n // _NW
    kern = pl.kernel(
        functools.partial(_sc_gather_body, n_per_w),
        out_type=jax.ShapeDtypeStruct((n, d), jnp.bfloat16),
        mesh=plsc.VectorSubcoreMesh(core_axis_name="c", subcore_axis_name="s"),
        scratch_types=[
            pltpu.VMEM((_CHUNK,), jnp.int32),
            pltpu.VMEM((_CHUNK, d), jnp.bfloat16),
            pltpu.SemaphoreType.DMA,
        ],
        compiler_params=pltpu.CompilerParams(use_tc_tiling_on_sc=False),
    )
    return kern(table, idx)


_TBK = 16384
_TB4 = _TBK // 4


def _tt_body(in_ref, out_ref):
    tin = in_ref[...]
    t = lax.dot_general(
        tin, jnp.eye(32, dtype=jnp.float32), (((0,), (0,)), ((), ())),
        preferred_element_type=jnp.float32)
    out_ref[...] = jnp.concatenate(
        [t[0:_TB4], t[_TB4:2 * _TB4], t[2 * _TB4:3 * _TB4], t[3 * _TB4:]],
        axis=1).astype(jnp.bfloat16)


def _tc_table_relayout(table_t):
    d, v = table_t.shape
    nstep = (v + _TBK - 1) // _TBK
    return pl.pallas_call(
        _tt_body,
        grid=(nstep,),
        in_specs=[pl.BlockSpec((d, _TBK), lambda i: (0, i))],
        out_specs=pl.BlockSpec((_TB4, 4 * d), lambda i: (i, 0)),
        out_shape=jax.ShapeDtypeStruct((nstep * _TB4, 4 * d), jnp.bfloat16),
    )(table_t)


def _row_ids(i):
    l = i & (_TBK - 1)
    return (i & ~(_TBK - 1)) + ((l & (_TB4 - 1)) << 2) + (l >> 12)


def _tc_body(scal_ref, tx_ref, coll_ref, m_ref,
             gamma_ref, beta_ref, out_ref):
    rb = tx_ref.shape[2]
    s3 = scal_ref[0]
    tx = tx_ref[0]
    iot = lax.broadcasted_iota(jnp.int32, (16, rb), 0)
    onehot_t = (tx == iot).astype(jnp.float32)
    f_t = jnp.concatenate(
        [s3, jnp.ones((1, rb), jnp.float32), onehot_t], axis=0)
    pre = lax.dot_general(
        f_t, m_ref[...], (((0,), (0,)), ((), ())),
        preferred_element_type=jnp.float32)
    c = coll_ref[...]
    coll = jnp.concatenate(
        [c[:, 0:32], c[:, 32:64], c[:, 64:96], c[:, 96:128]],
        axis=0).astype(jnp.float32)
    comb = jnp.concatenate(
        [pre[:, : _D_MODEL - 32], pre[:, _D_MODEL - 32:] + coll],
        axis=1)
    mu = jnp.mean(comb, axis=1, keepdims=True)
    dev = comb - mu
    var = jnp.mean(dev * dev, axis=1, keepdims=True)
    out_ref[0] = (dev * lax.rsqrt(var + _EPS) * gamma_ref[...]
                  + beta_ref[...])


def _tc_fused(scal, tx_t, coll_rows, m, gamma, beta, *,
              interpret=False):
    s, _, b = scal.shape
    full = lambda j: (0, 0)
    coll128 = coll_rows.reshape(s * b // 4, _D_MODEL)
    return pl.pallas_call(
        _tc_body,
        grid=(s,),
        in_specs=[
            pl.BlockSpec((1, 3, b), lambda j: (j, 0, 0)),
            pl.BlockSpec((1, 1, b), lambda j: (j, 0, 0)),
            pl.BlockSpec((b // 4, _D_MODEL), lambda j: (j, 0)),
            pl.BlockSpec(m.shape, full),
            pl.BlockSpec((1, _D_MODEL), full),
            pl.BlockSpec((1, _D_MODEL), full),
        ],
        out_specs=pl.BlockSpec((1, b, _D_MODEL), lambda j: (j, 0, 0)),
        out_shape=jax.ShapeDtypeStruct((s, b, _D_MODEL), jnp.float32),
        interpret=interpret,
    )(scal, tx_t.reshape(s, 1, b), coll128,
      m, gamma.reshape(1, -1), beta.reshape(1, -1))


def _assemble_m(W_value, b_value, W_gas, b_gas, W_vol, b_vol, tx_table):
    d4 = W_value.shape[1]
    d8 = W_gas.shape[1]
    m = jnp.zeros((20, _D_MODEL), jnp.float32)
    m = m.at[0, :d4].set(W_value[0])
    m = m.at[1, d4:d4 + d8].set(W_gas[0])
    m = m.at[2, d4 + d8:d4 + 2 * d8].set(W_vol[0])
    m = m.at[3, :d4].set(b_value)
    m = m.at[3, d4:d4 + d8].set(b_gas)
    m = m.at[3, d4 + d8:d4 + 2 * d8].set(b_vol)
    m = m.at[4:4 + tx_table.shape[0], d4 + 2 * d8:d4 + 2 * d8 + d4].set(tx_table)
    return m


def kernel(value, gas_fee, volume, tx_type, nft_collection,
           W_value, b_value, W_gas, b_gas, W_vol, b_vol,
           tx_table, coll_table, gamma, beta):
    b, s = value.shape
    n = b * s
    sc_idx = (_row_ids(nft_collection.T).reshape(s, 4, b // 4)
              .transpose(0, 2, 1).reshape(n))
    table_lin = _tc_table_relayout(coll_table.T)
    table32 = table_lin.reshape(table_lin.size // 32, 32)
    coll_rows = _sc_gather(table32, sc_idx)
    m = _assemble_m(W_value, b_value, W_gas, b_gas, W_vol, b_vol, tx_table)
    scal = jnp.stack([value.T, gas_fee.T, volume.T], axis=1)
    out_t = _tc_fused(scal, tx_type.T, coll_rows, m, gamma, beta)
    return out_t.transpose(1, 0, 2)


# speedup vs baseline: 1.4370x; 1.4370x over previous
import functools

import jax
import jax.numpy as jnp
from jax import lax
from jax.experimental import pallas as pl
from jax.experimental.pallas import tpu as pltpu
from jax.experimental.pallas import tpu_sc as plsc

_D_MODEL = 128
_EPS = 1e-5


_NC, _NS = 2, 16
_NW = _NC * _NS
_CHUNK = 1280


def _sc_gather_body(n_per_w, table_hbm, idx_hbm, out_hbm, idx_v, rows_v, sem):
    wid = lax.axis_index("s") * _NC + lax.axis_index("c")
    base = wid * n_per_w
    for j in range(n_per_w // _CHUNK):
        off = base + j * _CHUNK
        pltpu.sync_copy(idx_hbm.at[pl.ds(off, _CHUNK)], idx_v)
        pltpu.async_copy(table_hbm.at[idx_v], rows_v, sem).wait()
        pltpu.sync_copy(rows_v, out_hbm.at[pl.ds(off, _CHUNK)])


def _sc_gather(table, idx):
    n = idx.shape[0]
    d = table.shape[1]
    n_per_w = n // _NW
    kern = pl.kernel(
        functools.partial(_sc_gather_body, n_per_w),
        out_type=jax.ShapeDtypeStruct((n, d), jnp.float32),
        mesh=plsc.VectorSubcoreMesh(core_axis_name="c", subcore_axis_name="s"),
        scratch_types=[
            pltpu.VMEM((_CHUNK,), jnp.int32),
            pltpu.VMEM((_CHUNK, d), jnp.float32),
            pltpu.SemaphoreType.DMA,
        ],
        compiler_params=pltpu.CompilerParams(use_tc_tiling_on_sc=False),
    )
    return kern(table, idx)


_TBK = 32768
_TB4 = _TBK // 4


def _tt_body(in_ref, out_ref):
    tin = in_ref[...]
    t = lax.dot_general(
        tin, jnp.eye(32, dtype=jnp.float32), (((0,), (0,)), ((), ())),
        preferred_element_type=jnp.float32)
    out_ref[...] = jnp.concatenate(
        [t[0:_TB4], t[_TB4:2 * _TB4], t[2 * _TB4:3 * _TB4], t[3 * _TB4:]],
        axis=1)


def _tc_table_relayout(table_t):
    d, v = table_t.shape
    nstep = (v + _TBK - 1) // _TBK
    return pl.pallas_call(
        _tt_body,
        grid=(nstep,),
        in_specs=[pl.BlockSpec((d, _TBK), lambda i: (0, i))],
        out_specs=pl.BlockSpec((_TB4, 4 * d), lambda i: (i, 0)),
        out_shape=jax.ShapeDtypeStruct((nstep * _TB4, 4 * d), jnp.float32),
    )(table_t)


def _row_ids(i):
    l = i & (_TBK - 1)
    return (i & ~(_TBK - 1)) + ((l & (_TB4 - 1)) << 2) + (l >> 13)


def _tc_body(scal_ref, tx_ref, coll_ref, m_ref,
             gamma_ref, beta_ref, out_ref):
    rb = tx_ref.shape[2]
    s3 = scal_ref[0]
    tx = tx_ref[0]
    iot = lax.broadcasted_iota(jnp.int32, (16, rb), 0)
    onehot_t = (tx == iot).astype(jnp.float32)
    f_t = jnp.concatenate(
        [s3, jnp.ones((1, rb), jnp.float32), onehot_t], axis=0)
    pre = lax.dot_general(
        f_t, m_ref[...], (((0,), (0,)), ((), ())),
        preferred_element_type=jnp.float32)
    c = coll_ref[...]
    coll = jnp.concatenate(
        [c[:, 0:32], c[:, 32:64], c[:, 64:96], c[:, 96:128]], axis=0)
    comb = jnp.concatenate(
        [pre[:, : _D_MODEL - 32], pre[:, _D_MODEL - 32:] + coll],
        axis=1)
    mu = jnp.mean(comb, axis=1, keepdims=True)
    dev = comb - mu
    var = jnp.mean(dev * dev, axis=1, keepdims=True)
    out_ref[0] = (dev * lax.rsqrt(var + _EPS) * gamma_ref[...]
                  + beta_ref[...])


def _tc_fused(scal, tx_t, coll_rows, m, gamma, beta, *,
              interpret=False):
    s, _, b = scal.shape
    full = lambda j: (0, 0)
    coll128 = coll_rows.reshape(s * b // 4, _D_MODEL)
    return pl.pallas_call(
        _tc_body,
        grid=(s,),
        in_specs=[
            pl.BlockSpec((1, 3, b), lambda j: (j, 0, 0)),
            pl.BlockSpec((1, 1, b), lambda j: (j, 0, 0)),
            pl.BlockSpec((b // 4, _D_MODEL), lambda j: (j, 0)),
            pl.BlockSpec(m.shape, full),
            pl.BlockSpec((1, _D_MODEL), full),
            pl.BlockSpec((1, _D_MODEL), full),
        ],
        out_specs=pl.BlockSpec((1, b, _D_MODEL), lambda j: (j, 0, 0)),
        out_shape=jax.ShapeDtypeStruct((s, b, _D_MODEL), jnp.float32),
        interpret=interpret,
    )(scal, tx_t.reshape(s, 1, b), coll128,
      m, gamma.reshape(1, -1), beta.reshape(1, -1))


def _assemble_m(W_value, b_value, W_gas, b_gas, W_vol, b_vol, tx_table):
    d4 = W_value.shape[1]
    d8 = W_gas.shape[1]
    m = jnp.zeros((20, _D_MODEL), jnp.float32)
    m = m.at[0, :d4].set(W_value[0])
    m = m.at[1, d4:d4 + d8].set(W_gas[0])
    m = m.at[2, d4 + d8:d4 + 2 * d8].set(W_vol[0])
    m = m.at[3, :d4].set(b_value)
    m = m.at[3, d4:d4 + d8].set(b_gas)
    m = m.at[3, d4 + d8:d4 + 2 * d8].set(b_vol)
    m = m.at[4:4 + tx_table.shape[0], d4 + 2 * d8:d4 + 2 * d8 + d4].set(tx_table)
    return m


def kernel(value, gas_fee, volume, tx_type, nft_collection,
           W_value, b_value, W_gas, b_gas, W_vol, b_vol,
           tx_table, coll_table, gamma, beta):
    b, s = value.shape
    n = b * s
    sc_idx = (_row_ids(nft_collection.T).reshape(s, 4, b // 4)
              .transpose(0, 2, 1).reshape(n))
    table_lin = _tc_table_relayout(coll_table.T)
    table32 = table_lin.reshape(table_lin.size // 32, 32)
    coll_rows = _sc_gather(table32, sc_idx)
    m = _assemble_m(W_value, b_value, W_gas, b_gas, W_vol, b_vol, tx_table)
    scal = jnp.stack([value.T, gas_fee.T, volume.T], axis=1)
    out_t = _tc_fused(scal, tx_type.T, coll_rows, m, gamma, beta)
    return out_t.transpose(1, 0, 2)
